# baseline (speedup 1.0000x reference)
import functools

import jax
import jax.numpy as jnp
from jax.experimental import pallas as pl
from jax.experimental.pallas import tpu as pltpu
from jax.experimental.pallas import tpu_sc as plsc

Q = 1024
D = 16
K = 100000
Z = 8
TILE = 512
NTILES = (K + TILE - 1) // TILE
KPAD = NTILES * TILE
IDX_BIG = 2**30


NEG_BIG = -2.0e8


def _extract_top8(work, nids, exact_mask):
    vals, idxs = [], []
    for _ in range(Z):
        m = jnp.max(work, axis=0, keepdims=True)
        eq = work == m
        cand = jnp.where(eq, nids, NEG_BIG)
        si = jnp.max(cand, axis=0, keepdims=True)
        if exact_mask:
            work = jnp.where(cand == si, -jnp.inf, work)
        else:
            work = jnp.where(eq, -jnp.inf, work)
        vals.append(m)
        idxs.append(si)
    return jnp.concatenate(vals, axis=0), jnp.concatenate(idxs, axis=0)


def _topk_body(pt_ref, m_ref, out_ref, rv_ref, ri_ref, wk_ref, nid_ref,
               go_ref):
    t = pl.program_id(0)
    mtile = m_ref[0]
    dots = jax.lax.dot_general(
        mtile, pt_ref[...], (((1,), (0,)), ((), ())),
        preferred_element_type=jnp.float32)
    ids = jax.lax.broadcasted_iota(jnp.int32, (TILE, Q), 0) + t * TILE
    wk_ref[...] = jnp.where(ids < K, dots, -jnp.inf)
    go_ref[0] = 1

    @pl.when(t == 0)
    def _():
        nid_ref[...] = -jax.lax.broadcasted_iota(
            jnp.int32, (TILE, Q), 0).astype(jnp.float32)
        rv_ref[...] = jnp.full((Z, Q), -jnp.inf, jnp.float32)
        ri_ref[...] = jnp.full((Z, Q), NEG_BIG, jnp.float32)

    for _ in range(Z):
        @pl.when(go_ref[0] == 1)
        def _():
            wk = wk_ref[...]
            m = jnp.max(wk, axis=0, keepdims=True)
            eq = wk == m
            cand = jnp.where(eq, nid_ref[...], NEG_BIG)
            sl = jnp.max(cand, axis=0, keepdims=True)
            si = sl - jnp.float32(t * TILE)
            wk_ref[...] = jnp.where(eq, -jnp.inf, wk)
            rv = rv_ref[...]
            ri = ri_ref[...]
            nrv, nri = [], []
            gt_prev = None
            for s in range(Z):
                gt = m > rv[s:s + 1]
                if s == 0:
                    nrv.append(jnp.where(gt, m, rv[s:s + 1]))
                    nri.append(jnp.where(gt, si, ri[s:s + 1]))
                else:
                    nrv.append(jnp.where(
                        gt, jnp.where(gt_prev, rv[s - 1:s], m), rv[s:s + 1]))
                    nri.append(jnp.where(
                        gt, jnp.where(gt_prev, ri[s - 1:s], si), ri[s:s + 1]))
                gt_prev = gt
            rv_ref[...] = jnp.concatenate(nrv, axis=0)
            ri_ref[...] = jnp.concatenate(nri, axis=0)
            go_ref[0] = jnp.max(gt_prev.astype(jnp.int32))

    @pl.when(t == NTILES - 1)
    def _():
        out_ref[...] = (-ri_ref[...]).astype(jnp.int32)


def _topk_indices(pt, m3):
    return pl.pallas_call(
        _topk_body,
        grid=(NTILES,),
        in_specs=[
            pl.BlockSpec((D, Q), lambda t: (0, 0)),
            pl.BlockSpec((1, TILE, D), lambda t: (t, 0, 0)),
        ],
        out_specs=pl.BlockSpec((Z, Q), lambda t: (0, 0)),
        out_shape=jax.ShapeDtypeStruct((Z, Q), jnp.int32),
        scratch_shapes=[
            pltpu.VMEM((Z, Q), jnp.float32),
            pltpu.VMEM((Z, Q), jnp.float32),
            pltpu.VMEM((TILE, Q), jnp.float32),
            pltpu.VMEM((TILE, Q), jnp.float32),
            pltpu.SMEM((1,), jnp.int32),
        ],
        compiler_params=pltpu.CompilerParams(
            dimension_semantics=("arbitrary",)),
    )(pt, m3)


_SC_CORES = 2
_SC_SUBCORES = 16
_SC_WORKERS = _SC_CORES * _SC_SUBCORES


def _sc_gather(mem_flat, eidx):
    n = Z * Q * D
    per_w = n // _SC_WORKERS
    mesh = plsc.VectorSubcoreMesh(core_axis_name="c", subcore_axis_name="s")

    @functools.partial(
        pl.kernel,
        out_type=jax.ShapeDtypeStruct((n,), jnp.float32),
        mesh=mesh,
        scratch_types=[
            pltpu.VMEM((per_w,), jnp.int32),
            pltpu.VMEM((per_w,), jnp.float32),
            pltpu.SemaphoreType.DMA,
        ],
    )
    def _k(table_hbm, idx_hbm, out_hbm, idx_v, rows_v, sem):
        wid = jax.lax.axis_index("s") * _SC_CORES + jax.lax.axis_index("c")
        base = wid * per_w
        pltpu.sync_copy(idx_hbm.at[pl.ds(base, per_w)], idx_v)
        pltpu.async_copy(table_hbm.at[idx_v], rows_v, sem).wait()
        pltpu.sync_copy(rows_v, out_hbm.at[pl.ds(base, per_w)])

    return _k(mem_flat, eidx)


def _loss_body(g_ref, p_ref, o_ref):
    acc = g_ref[0]
    for z in range(1, Z):
        acc = acc + g_ref[z]
    err = acc * jnp.float32(1.0 / Z) - p_ref[...]
    sq = jnp.sum(err * err, axis=(0, 1), keepdims=True)
    o_ref[...] = sq * jnp.float32(1.0 / (Q * D))


def _mse_loss(g3, p):
    return pl.pallas_call(
        _loss_body,
        in_specs=[
            pl.BlockSpec((Z, Q, D), lambda: (0, 0, 0)),
            pl.BlockSpec((Q, D), lambda: (0, 0)),
        ],
        out_specs=pl.BlockSpec((1, 1), lambda: (0, 0)),
        out_shape=jax.ShapeDtypeStruct((1, 1), jnp.float32),
    )(g3, p)


def kernel(prototypes, memory_points):
    mp = jnp.pad(memory_points, ((0, KPAD - K), (0, 0)))
    m3 = mp.reshape(NTILES, TILE, D)
    idx = _topk_indices(prototypes.T, m3)
    eidx = (idx.reshape(Z * Q, 1) * D + jnp.arange(D, dtype=jnp.int32))
    g = _sc_gather(memory_points.reshape(K * D), eidx.reshape(Z * Q * D))
    loss = _mse_loss(g.reshape(Z, Q, D), prototypes)
    return loss[0, 0]


# speedup vs baseline: 1.0402x; 1.0402x over previous
import functools

import jax
import jax.numpy as jnp
from jax.experimental import pallas as pl
from jax.experimental.pallas import tpu as pltpu
from jax.experimental.pallas import tpu_sc as plsc

Q = 1024
D = 16
K = 100000
Z = 8
TILE = 1024
NTILES = (K + TILE - 1) // TILE
KPAD = NTILES * TILE
IDX_BIG = 2**30


NEG_BIG = -2.0e8


def _extract_top8(work, nids, exact_mask):
    vals, idxs = [], []
    for _ in range(Z):
        m = jnp.max(work, axis=0, keepdims=True)
        eq = work == m
        cand = jnp.where(eq, nids, NEG_BIG)
        si = jnp.max(cand, axis=0, keepdims=True)
        if exact_mask:
            work = jnp.where(cand == si, -jnp.inf, work)
        else:
            work = jnp.where(eq, -jnp.inf, work)
        vals.append(m)
        idxs.append(si)
    return jnp.concatenate(vals, axis=0), jnp.concatenate(idxs, axis=0)


def _topk_body(pt_ref, m_ref, out_ref, rv_ref, ri_ref, wk_ref, nid_ref,
               go_ref):
    t = pl.program_id(0)
    mtile = m_ref[0]
    dots = jax.lax.dot_general(
        mtile, pt_ref[...], (((1,), (0,)), ((), ())),
        preferred_element_type=jnp.float32)
    ids = jax.lax.broadcasted_iota(jnp.int32, (TILE, Q), 0) + t * TILE
    wk_ref[...] = jnp.where(ids < K, dots, -jnp.inf)
    go_ref[0] = 1

    @pl.when(t == 0)
    def _():
        nid_ref[...] = -jax.lax.broadcasted_iota(
            jnp.int32, (TILE, Q), 0).astype(jnp.float32)
        rv_ref[...] = jnp.full((Z, Q), -jnp.inf, jnp.float32)
        ri_ref[...] = jnp.full((Z, Q), NEG_BIG, jnp.float32)

    for _ in range(Z):
        @pl.when(go_ref[0] == 1)
        def _():
            wk = wk_ref[...]
            m = jnp.max(wk, axis=0, keepdims=True)
            eq = wk == m
            cand = jnp.where(eq, nid_ref[...], NEG_BIG)
            sl = jnp.max(cand, axis=0, keepdims=True)
            si = sl - jnp.float32(t * TILE)
            wk_ref[...] = jnp.where(eq, -jnp.inf, wk)
            rv = rv_ref[...]
            ri = ri_ref[...]
            nrv, nri = [], []
            gt_prev = None
            for s in range(Z):
                gt = m > rv[s:s + 1]
                if s == 0:
                    nrv.append(jnp.where(gt, m, rv[s:s + 1]))
                    nri.append(jnp.where(gt, si, ri[s:s + 1]))
                else:
                    nrv.append(jnp.where(
                        gt, jnp.where(gt_prev, rv[s - 1:s], m), rv[s:s + 1]))
                    nri.append(jnp.where(
                        gt, jnp.where(gt_prev, ri[s - 1:s], si), ri[s:s + 1]))
                gt_prev = gt
            rv_ref[...] = jnp.concatenate(nrv, axis=0)
            ri_ref[...] = jnp.concatenate(nri, axis=0)
            go_ref[0] = jnp.max(gt_prev.astype(jnp.int32))

    @pl.when(t == NTILES - 1)
    def _():
        out_ref[...] = (-ri_ref[...]).astype(jnp.int32)


def _topk_indices(pt, m3):
    return pl.pallas_call(
        _topk_body,
        grid=(NTILES,),
        in_specs=[
            pl.BlockSpec((D, Q), lambda t: (0, 0)),
            pl.BlockSpec((1, TILE, D), lambda t: (t, 0, 0)),
        ],
        out_specs=pl.BlockSpec((Z, Q), lambda t: (0, 0)),
        out_shape=jax.ShapeDtypeStruct((Z, Q), jnp.int32),
        scratch_shapes=[
            pltpu.VMEM((Z, Q), jnp.float32),
            pltpu.VMEM((Z, Q), jnp.float32),
            pltpu.VMEM((TILE, Q), jnp.float32),
            pltpu.VMEM((TILE, Q), jnp.float32),
            pltpu.SMEM((1,), jnp.int32),
        ],
        compiler_params=pltpu.CompilerParams(
            dimension_semantics=("arbitrary",)),
    )(pt, m3)


_SC_CORES = 2
_SC_SUBCORES = 16
_SC_WORKERS = _SC_CORES * _SC_SUBCORES


def _sc_gather(mem_flat, eidx):
    n = Z * Q * D
    per_w = n // _SC_WORKERS
    mesh = plsc.VectorSubcoreMesh(core_axis_name="c", subcore_axis_name="s")

    @functools.partial(
        pl.kernel,
        out_type=jax.ShapeDtypeStruct((n,), jnp.float32),
        mesh=mesh,
        scratch_types=[
            pltpu.VMEM((per_w,), jnp.int32),
            pltpu.VMEM((per_w,), jnp.float32),
            pltpu.SemaphoreType.DMA,
        ],
    )
    def _k(table_hbm, idx_hbm, out_hbm, idx_v, rows_v, sem):
        wid = jax.lax.axis_index("s") * _SC_CORES + jax.lax.axis_index("c")
        base = wid * per_w
        pltpu.sync_copy(idx_hbm.at[pl.ds(base, per_w)], idx_v)
        pltpu.async_copy(table_hbm.at[idx_v], rows_v, sem).wait()
        pltpu.sync_copy(rows_v, out_hbm.at[pl.ds(base, per_w)])

    return _k(mem_flat, eidx)


def _loss_body(g_ref, p_ref, o_ref):
    acc = g_ref[0]
    for z in range(1, Z):
        acc = acc + g_ref[z]
    err = acc * jnp.float32(1.0 / Z) - p_ref[...]
    sq = jnp.sum(err * err, axis=(0, 1), keepdims=True)
    o_ref[...] = sq * jnp.float32(1.0 / (Q * D))


def _mse_loss(g3, p):
    return pl.pallas_call(
        _loss_body,
        in_specs=[
            pl.BlockSpec((Z, Q, D), lambda: (0, 0, 0)),
            pl.BlockSpec((Q, D), lambda: (0, 0)),
        ],
        out_specs=pl.BlockSpec((1, 1), lambda: (0, 0)),
        out_shape=jax.ShapeDtypeStruct((1, 1), jnp.float32),
    )(g3, p)


def kernel(prototypes, memory_points):
    mp = jnp.pad(memory_points, ((0, KPAD - K), (0, 0)))
    m3 = mp.reshape(NTILES, TILE, D)
    idx = _topk_indices(prototypes.T, m3)
    eidx = (idx.reshape(Z * Q, 1) * D + jnp.arange(D, dtype=jnp.int32))
    g = _sc_gather(memory_points.reshape(K * D), eidx.reshape(Z * Q * D))
    loss = _mse_loss(g.reshape(Z, Q, D), prototypes)
    return loss[0, 0]


# speedup vs baseline: 1.0677x; 1.0264x over previous
import functools

import jax
import jax.numpy as jnp
from jax.experimental import pallas as pl
from jax.experimental.pallas import tpu as pltpu
from jax.experimental.pallas import tpu_sc as plsc

Q = 1024
D = 16
K = 100000
Z = 8
TILE = 1024
NTILES = (K + TILE - 1) // TILE
KPAD = NTILES * TILE
IDX_BIG = 2**30


NEG_BIG = -2.0e8


def _extract_top8(work, nids, exact_mask):
    vals, idxs = [], []
    for _ in range(Z):
        m = jnp.max(work, axis=0, keepdims=True)
        eq = work == m
        cand = jnp.where(eq, nids, NEG_BIG)
        si = jnp.max(cand, axis=0, keepdims=True)
        if exact_mask:
            work = jnp.where(cand == si, -jnp.inf, work)
        else:
            work = jnp.where(eq, -jnp.inf, work)
        vals.append(m)
        idxs.append(si)
    return jnp.concatenate(vals, axis=0), jnp.concatenate(idxs, axis=0)


def _topk_body(pt_ref, m_ref, out_ref, rv_ref, ri_ref, wk_ref, nid_ref,
               go_ref):
    t = pl.program_id(0)
    mtile = m_ref[...]
    dots = jax.lax.dot_general(
        mtile, pt_ref[...], (((1,), (0,)), ((), ())),
        preferred_element_type=jnp.float32)
    ids = jax.lax.broadcasted_iota(jnp.int32, (TILE, Q), 0) + t * TILE
    wk_ref[...] = jnp.where(ids < K, dots, -jnp.inf)
    go_ref[0] = 1

    @pl.when(t == 0)
    def _():
        nid_ref[...] = -jax.lax.broadcasted_iota(
            jnp.int32, (TILE, Q), 0).astype(jnp.float32)
        rv_ref[...] = jnp.full((Z, Q), -jnp.inf, jnp.float32)
        ri_ref[...] = jnp.full((Z, Q), NEG_BIG, jnp.float32)

    for _ in range(Z):
        @pl.when(go_ref[0] == 1)
        def _():
            wk = wk_ref[...]
            m = jnp.max(wk, axis=0, keepdims=True)
            eq = wk == m
            cand = jnp.where(eq, nid_ref[...], NEG_BIG)
            sl = jnp.max(cand, axis=0, keepdims=True)
            si = sl - jnp.float32(t * TILE)
            wk_ref[...] = jnp.where(eq, -jnp.inf, wk)
            rv = rv_ref[...]
            ri = ri_ref[...]
            nrv, nri = [], []
            gt_prev = None
            for s in range(Z):
                gt = m > rv[s:s + 1]
                if s == 0:
                    nrv.append(jnp.where(gt, m, rv[s:s + 1]))
                    nri.append(jnp.where(gt, si, ri[s:s + 1]))
                else:
                    nrv.append(jnp.where(
                        gt, jnp.where(gt_prev, rv[s - 1:s], m), rv[s:s + 1]))
                    nri.append(jnp.where(
                        gt, jnp.where(gt_prev, ri[s - 1:s], si), ri[s:s + 1]))
                gt_prev = gt
            rv_ref[...] = jnp.concatenate(nrv, axis=0)
            ri_ref[...] = jnp.concatenate(nri, axis=0)
            go_ref[0] = jnp.max(gt_prev.astype(jnp.int32))

    @pl.when(t == NTILES - 1)
    def _():
        gidx = (-ri_ref[...]).astype(jnp.int32)
        out_ref[...] = (gidx[:, :, None] * D
                        + jax.lax.broadcasted_iota(jnp.int32, (Z, Q, D), 2))


def _topk_indices(pt, mem):
    return pl.pallas_call(
        _topk_body,
        grid=(NTILES,),
        in_specs=[
            pl.BlockSpec((D, Q), lambda t: (0, 0)),
            pl.BlockSpec((TILE, D), lambda t: (t, 0)),
        ],
        out_specs=pl.BlockSpec((Z, Q, D), lambda t: (0, 0, 0)),
        out_shape=jax.ShapeDtypeStruct((Z, Q, D), jnp.int32),
        scratch_shapes=[
            pltpu.VMEM((Z, Q), jnp.float32),
            pltpu.VMEM((Z, Q), jnp.float32),
            pltpu.VMEM((TILE, Q), jnp.float32),
            pltpu.VMEM((TILE, Q), jnp.float32),
            pltpu.SMEM((1,), jnp.int32),
        ],
        compiler_params=pltpu.CompilerParams(
            dimension_semantics=("arbitrary",)),
    )(pt, mem)


_SC_CORES = 2
_SC_SUBCORES = 16
_SC_WORKERS = _SC_CORES * _SC_SUBCORES


def _sc_gather(mem_flat, eidx):
    n = Z * Q * D
    per_w = n // _SC_WORKERS
    mesh = plsc.VectorSubcoreMesh(core_axis_name="c", subcore_axis_name="s")

    @functools.partial(
        pl.kernel,
        out_type=jax.ShapeDtypeStruct((n,), jnp.float32),
        mesh=mesh,
        scratch_types=[
            pltpu.VMEM((per_w,), jnp.int32),
            pltpu.VMEM((per_w,), jnp.float32),
            pltpu.SemaphoreType.DMA,
        ],
    )
    def _k(table_hbm, idx_hbm, out_hbm, idx_v, rows_v, sem):
        wid = jax.lax.axis_index("s") * _SC_CORES + jax.lax.axis_index("c")
        base = wid * per_w
        pltpu.sync_copy(idx_hbm.at[pl.ds(base, per_w)], idx_v)
        pltpu.async_copy(table_hbm.at[idx_v], rows_v, sem).wait()
        pltpu.sync_copy(rows_v, out_hbm.at[pl.ds(base, per_w)])

    return _k(mem_flat, eidx)


def _loss_body(g_ref, p_ref, o_ref):
    acc = g_ref[0]
    for z in range(1, Z):
        acc = acc + g_ref[z]
    err = acc * jnp.float32(1.0 / Z) - p_ref[...]
    sq = jnp.sum(err * err, axis=(0, 1), keepdims=True)
    o_ref[...] = sq * jnp.float32(1.0 / (Q * D))


def _mse_loss(g3, p):
    return pl.pallas_call(
        _loss_body,
        in_specs=[
            pl.BlockSpec((Z, Q, D), lambda: (0, 0, 0)),
            pl.BlockSpec((Q, D), lambda: (0, 0)),
        ],
        out_specs=pl.BlockSpec((1, 1), lambda: (0, 0)),
        out_shape=jax.ShapeDtypeStruct((1, 1), jnp.float32),
    )(g3, p)


def kernel(prototypes, memory_points):
    eidx = _topk_indices(prototypes.T, memory_points)
    g = _sc_gather(memory_points.reshape(K * D), eidx.reshape(Z * Q * D))
    loss = _mse_loss(g.reshape(Z, Q, D), prototypes)
    return loss[0, 0]


# speedup vs baseline: 1.0680x; 1.0002x over previous
import functools

import jax
import jax.numpy as jnp
from jax.experimental import pallas as pl
from jax.experimental.pallas import tpu as pltpu
from jax.experimental.pallas import tpu_sc as plsc

Q = 1024
D = 16
K = 100000
Z = 8
TILE = 1024
NTILES = (K + TILE - 1) // TILE
NEG_BIG = -2.0e8


def _topk_body(pt_ref, m_ref, out_ref, rv_ref, ri_ref, wk_ref, nid_ref,
               go_ref):
    t = pl.program_id(0)
    mtile = m_ref[...]
    dots = jax.lax.dot_general(
        mtile, pt_ref[...], (((1,), (0,)), ((), ())),
        preferred_element_type=jnp.float32)
    ids = jax.lax.broadcasted_iota(jnp.int32, (TILE, Q), 0) + t * TILE
    wk_ref[...] = jnp.where(ids < K, dots, -jnp.inf)
    go_ref[0] = 1

    @pl.when(t == 0)
    def _():
        nid_ref[...] = -jax.lax.broadcasted_iota(
            jnp.int32, (TILE, Q), 0).astype(jnp.float32)
        rv_ref[...] = jnp.full((Z, Q), -jnp.inf, jnp.float32)
        ri_ref[...] = jnp.full((Z, Q), NEG_BIG, jnp.float32)

    for _ in range(Z):
        @pl.when(go_ref[0] == 1)
        def _():
            wk = wk_ref[...]
            m = jnp.max(wk, axis=0, keepdims=True)
            eq = wk == m
            cand = jnp.where(eq, nid_ref[...], NEG_BIG)
            sl = jnp.max(cand, axis=0, keepdims=True)
            si = sl - jnp.float32(t * TILE)
            wk_ref[...] = jnp.where(eq, -jnp.inf, wk)
            rv = rv_ref[...]
            ri = ri_ref[...]
            nrv, nri = [], []
            gt_prev = None
            for s in range(Z):
                gt = m > rv[s:s + 1]
                if s == 0:
                    nrv.append(jnp.where(gt, m, rv[s:s + 1]))
                    nri.append(jnp.where(gt, si, ri[s:s + 1]))
                else:
                    nrv.append(jnp.where(
                        gt, jnp.where(gt_prev, rv[s - 1:s], m), rv[s:s + 1]))
                    nri.append(jnp.where(
                        gt, jnp.where(gt_prev, ri[s - 1:s], si), ri[s:s + 1]))
                gt_prev = gt
            rv_ref[...] = jnp.concatenate(nrv, axis=0)
            ri_ref[...] = jnp.concatenate(nri, axis=0)
            go_ref[0] = jnp.max(gt_prev.astype(jnp.int32))

    @pl.when(t == NTILES - 1)
    def _():
        gidx = (-ri_ref[...]).astype(jnp.int32)
        out_ref[...] = (gidx[:, :, None] * D
                        + jax.lax.broadcasted_iota(jnp.int32, (Z, Q, D), 2))


def _topk_indices(pt, mem):
    return pl.pallas_call(
        _topk_body,
        grid=(NTILES,),
        in_specs=[
            pl.BlockSpec((D, Q), lambda t: (0, 0)),
            pl.BlockSpec((TILE, D), lambda t: (t, 0)),
        ],
        out_specs=pl.BlockSpec((Z, Q, D), lambda t: (0, 0, 0)),
        out_shape=jax.ShapeDtypeStruct((Z, Q, D), jnp.int32),
        scratch_shapes=[
            pltpu.VMEM((Z, Q), jnp.float32),
            pltpu.VMEM((Z, Q), jnp.float32),
            pltpu.VMEM((TILE, Q), jnp.float32),
            pltpu.VMEM((TILE, Q), jnp.float32),
            pltpu.SMEM((1,), jnp.int32),
        ],
        compiler_params=pltpu.CompilerParams(
            dimension_semantics=("arbitrary",)),
    )(pt, mem)


_SC_CORES = 2
_SC_SUBCORES = 16
_SC_WORKERS = _SC_CORES * _SC_SUBCORES


def _sc_gather(mem_flat, eidx):
    n = Z * Q * D
    per_w = n // _SC_WORKERS
    mesh = plsc.VectorSubcoreMesh(core_axis_name="c", subcore_axis_name="s")

    @functools.partial(
        pl.kernel,
        out_type=jax.ShapeDtypeStruct((n,), jnp.float32),
        mesh=mesh,
        scratch_types=[
            pltpu.VMEM((per_w,), jnp.int32),
            pltpu.VMEM((per_w,), jnp.float32),
            pltpu.SemaphoreType.DMA,
        ],
    )
    def _k(table_hbm, idx_hbm, out_hbm, idx_v, rows_v, sem):
        wid = jax.lax.axis_index("s") * _SC_CORES + jax.lax.axis_index("c")
        base = wid * per_w
        pltpu.sync_copy(idx_hbm.at[pl.ds(base, per_w)], idx_v)
        pltpu.async_copy(table_hbm.at[idx_v], rows_v, sem).wait()
        pltpu.sync_copy(rows_v, out_hbm.at[pl.ds(base, per_w)])

    return _k(mem_flat, eidx)


def _loss_body(g_ref, p_ref, o_ref):
    acc = g_ref[0]
    for z in range(1, Z):
        acc = acc + g_ref[z]
    err = acc * jnp.float32(1.0 / Z) - p_ref[...]
    sq = jnp.sum(err * err, axis=(0, 1), keepdims=True)
    o_ref[...] = sq * jnp.float32(1.0 / (Q * D))


def _mse_loss(g3, p):
    return pl.pallas_call(
        _loss_body,
        in_specs=[
            pl.BlockSpec((Z, Q, D), lambda: (0, 0, 0)),
            pl.BlockSpec((Q, D), lambda: (0, 0)),
        ],
        out_specs=pl.BlockSpec((1, 1), lambda: (0, 0)),
        out_shape=jax.ShapeDtypeStruct((1, 1), jnp.float32),
    )(g3, p)


def kernel(prototypes, memory_points):
    eidx = _topk_indices(prototypes.T, memory_points)
    g = _sc_gather(memory_points.reshape(K * D), eidx.reshape(Z * Q * D))
    loss = _mse_loss(g.reshape(Z, Q, D), prototypes)
    return loss[0, 0]
